# baseline (speedup 1.0000x reference)
import jax
import jax.numpy as jnp
from jax.experimental import pallas as pl


def _body(m_ref, x_ref, o_ref):
    o_ref[...] = x_ref[...] * m_ref[...]


def kernel(input, mask):
    B, C, K, H, W = input.shape
    BC = B * C
    HW = H * W
    x = input.reshape(BC, K, HW)
    m = mask.reshape(BC, 1, HW)

    HWB = 12544
    nl = HW // HWB

    out = pl.pallas_call(
        _body,
        grid=(BC, nl),
        in_specs=[
            pl.BlockSpec((1, 1, HWB), lambda i, l: (i, 0, l)),
            pl.BlockSpec((1, K, HWB), lambda i, l: (i, 0, l)),
        ],
        out_specs=pl.BlockSpec((1, K, HWB), lambda i, l: (i, 0, l)),
        out_shape=jax.ShapeDtypeStruct((BC, K, HW), x.dtype),
    )(m, x)
    return out.reshape(B, C, K, H, W)
